# Initial kernel scaffold; baseline (speedup 1.0000x reference)
import jax
import jax.numpy as jnp
from jax.experimental import pallas as pl


def kernel(user_table, item_table, edge_index_iu, edge_index_ui, pos_items_ur, neg_items_ur):
    raise NotImplementedError("write your pallas kernel here")


# speedup vs baseline: 3.7663x; 3.7663x over previous
import functools

import jax
import jax.numpy as jnp
from jax import lax
from jax.experimental import pallas as pl
from jax.experimental.pallas import tpu as pltpu
from jax.experimental.pallas import tpu_sc as plsc

N_NODES = 10000
D = 128
NE = 320000
NC = 2
NS = 16
NW = NC * NS
NPAD = 10240
RPW = NPAD // NW
EB = 32
LROWS = 680
SB = 3200
CH = 64
DECAY = 1e-4
EPS = 1e-16

_mesh = plsc.VectorSubcoreMesh(core_axis_name="c", subcore_axis_name="s")

_f32 = jnp.float32
_i32 = jnp.int32

_GDN = lax.GatherDimensionNumbers(
    offset_dims=(), collapsed_slice_dims=(0,), start_index_map=(0,))


def _lperm(x, idx):
    return lax.gather(x, idx.reshape(16, 1), _GDN, slice_sizes=(1,),
                      mode=lax.GatherScatterMode.PROMISE_IN_BOUNDS)


def _hsum(x):
    lanes = lax.iota(_i32, 16)
    for sh in (8, 4, 2, 1):
        x = x + _lperm(x, lanes ^ sh)
    return x


def _wid():
    return lax.axis_index("s") * NC + lax.axis_index("c")


@functools.partial(
    pl.kernel,
    out_type=(jax.ShapeDtypeStruct((NW, LROWS, 16), _i32),
              jax.ShapeDtypeStruct((NW, 16), _i32)),
    mesh=_mesh,
    scratch_types=[
        pltpu.VMEM((SB,), _i32),
        pltpu.VMEM((SB,), _i32),
        pltpu.VMEM((LROWS, 16), _i32),
        pltpu.VMEM((16,), _i32),
    ],
)
def _partition(src_idx, dst_idx, lists, cnt, sstage, dstage, pbuf, cvec):
    wid = _wid()
    lo = wid * RPW
    hi = lo + RPW
    lanes = lax.iota(_i32, 16)
    one_i = jnp.full((16,), 1, _i32)
    zero_i = jnp.full((16,), 0, _i32)
    last = jnp.full((16,), 15, _i32)
    pad_row = jnp.full((16,), hi, _i32)

    def fill(i, carry):
        pbuf[i, pl.ds(0, 16)] = pad_row
        return carry

    lax.fori_loop(0, LROWS, fill, 0)

    def step(it, state):
        off = it * SB
        pltpu.sync_copy(src_idx.at[pl.ds(off, SB)], sstage)
        pltpu.sync_copy(dst_idx.at[pl.ds(off, SB)], dstage)

        def group(j, st):
            r, f, pend = st
            sl = pl.ds(j * 16, 16)
            dg = dstage[sl]
            sg = sstage[sl]
            m = (dg >= lo) & (dg < hi)
            rank = jnp.where(m, one_i, zero_i)
            for sh in (1, 2, 4, 8):
                shifted = _lperm(rank, jnp.maximum(lanes - sh, 0))
                rank = rank + jnp.where(lanes >= sh, shifted, zero_i)
            target = lanes + 1
            sel = zero_i
            for bit in (8, 4, 2, 1):
                probe = _lperm(rank, sel + (bit - 1))
                sel = jnp.where(probe >= target, sel, sel + bit)
            packed = (sg << 14) | dg
            comp = _lperm(packed, sel)
            npop = rank[15]
            merge_a = jnp.where(lanes < f, pend,
                                _lperm(comp, jnp.maximum(lanes - f, 0)))
            merge_b = _lperm(comp, jnp.minimum(lanes + (16 - f), last))
            pbuf[r, pl.ds(0, 16)] = merge_a
            pbuf[r + 1, pl.ds(0, 16)] = merge_b
            tot = f + npop
            roll = tot >> 4
            rollv = jnp.full((16,), roll, _i32)
            pend = merge_a * (1 - rollv) + merge_b * rollv
            return (r + roll, tot & 15, pend)

        return lax.fori_loop(0, SB // 16, group, state)

    r, f, _ = lax.fori_loop(0, NE // SB, step,
                            (jnp.asarray(0, _i32), jnp.asarray(0, _i32),
                             pad_row))
    pbuf[r, pl.ds(0, 16)] = jnp.where(lanes < f,
                                      pbuf[r, pl.ds(0, 16)], pad_row)
    pbuf[r + 1, pl.ds(0, 16)] = pad_row
    pltpu.sync_copy(pbuf, lists.at[wid])
    cvec[...] = jnp.full((16,), r * 16 + f, _i32)
    pltpu.sync_copy(cvec, cnt.at[wid])


@functools.partial(
    pl.kernel,
    out_type=jax.ShapeDtypeStruct((NPAD, D), _f32),
    mesh=_mesh,
    scratch_types=[
        pltpu.VMEM((EB,), _i32),
        pltpu.VMEM((EB,), _i32),
        pltpu.VMEM((EB,), _i32),
        pltpu.VMEM((EB, D), _f32),
        pltpu.VMEM((RPW + 8, D), _f32),
        pltpu.VMEM((RPW + 8, D), _f32),
        pltpu.VMEM((RPW + 8, 16), _f32),
        pltpu.VMEM((16,), _i32),
        pltpu.SemaphoreType.DMA,
    ],
)
def _conv(src_tab, dst_tab, lists, cnt, out,
          pk, lbuf, sidx, srows, dloc, num, den, cvec, sem):
    wid = _wid()
    lo = wid * RPW
    zero16 = jnp.full((16,), 0.0, _f32)

    pltpu.sync_copy(dst_tab.at[pl.ds(lo, RPW)], dloc.at[pl.ds(0, RPW)])

    def zrow(r, carry):
        for q in range(8):
            num[r, pl.ds(q * 16, 16)] = zero16
        den[r, pl.ds(0, 16)] = zero16
        return carry

    lax.fori_loop(0, RPW + 8, zrow, 0)
    for r in range(8):
        for q in range(8):
            dloc[RPW + r, pl.ds(q * 16, 16)] = zero16

    pltpu.sync_copy(cnt.at[wid], cvec)
    cvec[...] = (cvec[...] + (EB - 1)) >> 5
    nblk = cvec[...][0]

    def edge(k, lr, sv):
        acc = sv[0] * dloc[lr, pl.ds(0, 16)]
        for q in range(1, 8):
            acc += sv[q] * dloc[lr, pl.ds(q * 16, 16)]
        e = _hsum(acc)
        e = jnp.where(e >= 0.0, e, 0.2 * e)
        w = jnp.exp(e)
        for q in range(8):
            sl = pl.ds(q * 16, 16)
            num[lr, sl] = num[lr, sl] + w * sv[q]
        den[lr, pl.ds(0, 16)] = den[lr, pl.ds(0, 16)] + w

    def block(b, carry):
        base = b * EB
        pltpu.sync_copy(lists.at[wid, pl.ds(base, EB)], pk)
        for g in range(EB // 16):
            p = pk[pl.ds(g * 16, 16)]
            sidx[pl.ds(g * 16, 16)] = p >> 14
            lbuf[pl.ds(g * 16, 16)] = (p & 16383) - lo
        pltpu.async_copy(src_tab.at[sidx], srows, sem).wait()
        for g in range(EB // 16):
            lg = lbuf[pl.ds(g * 16, 16)]
            for j in range(16):
                k = g * 16 + j
                edge(k, lg[j],
                     [srows[k, pl.ds(q * 16, 16)] for q in range(8)])
        return carry

    lax.fori_loop(0, nblk, block, 0)

    def nrow(r, carry):
        inv = 1.0 / (den[r, pl.ds(0, 16)] + EPS)
        for q in range(8):
            sl = pl.ds(q * 16, 16)
            num[r, sl] = num[r, sl] * inv
        return carry

    lax.fori_loop(0, RPW, nrow, 0)
    pltpu.sync_copy(num.at[pl.ds(0, RPW)], out.at[pl.ds(lo, RPW)])


@functools.partial(
    pl.kernel,
    out_type=jax.ShapeDtypeStruct((NPAD, D), _f32),
    mesh=_mesh,
    scratch_types=[
        pltpu.VMEM((CH, D), _f32),
        pltpu.VMEM((CH, D), _f32),
        pltpu.VMEM((CH, D), _f32),
    ],
)
def _mean3(a, b, c_tab, out, ta, tb, tc):
    wid = _wid()
    third = jnp.full((16,), 1.0 / 3.0, _f32)

    def chunk(ch, carry):
        base = wid * RPW + ch * CH
        pltpu.sync_copy(a.at[pl.ds(base, CH)], ta)
        pltpu.sync_copy(b.at[pl.ds(base, CH)], tb)
        pltpu.sync_copy(c_tab.at[pl.ds(base, CH)], tc)

        def row(r, c2):
            for q in range(8):
                sl = pl.ds(q * 16, 16)
                ta[r, sl] = (ta[r, sl] + tb[r, sl] + tc[r, sl]) * third
            return c2

        lax.fori_loop(0, CH, row, 0)
        pltpu.sync_copy(ta, out.at[pl.ds(base, CH)])
        return carry

    lax.fori_loop(0, RPW // CH, chunk, 0)


@functools.partial(
    pl.kernel,
    out_type=(jax.ShapeDtypeStruct((NPAD, D), _f32),
              jax.ShapeDtypeStruct((NPAD, D), _f32)),
    mesh=_mesh,
    scratch_types=[
        pltpu.VMEM((CH,), _i32),
        pltpu.VMEM((CH,), _i32),
        pltpu.VMEM((CH, D), _f32),
        pltpu.VMEM((CH, D), _f32),
        pltpu.SemaphoreType.DMA,
        pltpu.SemaphoreType.DMA,
    ],
)
def _pair_gather(imean, posp, negp, pg, ng, idxp, idxn, prow, nrow, semp, semn):
    wid = _wid()

    def chunk(ch, carry):
        base = wid * RPW + ch * CH
        pltpu.sync_copy(posp.at[pl.ds(base, CH)], idxp)
        pltpu.sync_copy(negp.at[pl.ds(base, CH)], idxn)
        cp = pltpu.async_copy(imean.at[idxp], prow, semp)
        cn = pltpu.async_copy(imean.at[idxn], nrow, semn)
        cp.wait()
        cn.wait()
        pltpu.sync_copy(prow, pg.at[pl.ds(base, CH)])
        pltpu.sync_copy(nrow, ng.at[pl.ds(base, CH)])
        return carry

    lax.fori_loop(0, RPW // CH, chunk, 0)


def _loss_body(u0, u1, u2, pg, ng, mf_ref, el_ref):
    um = (u0[...] + u1[...] + u2[...]) * (1.0 / 3.0)
    p = pg[...]
    n = ng[...]
    ps = jnp.sum(um * p, axis=1, keepdims=True)
    ns = jnp.sum(um * n, axis=1, keepdims=True)
    x = ns - ps
    valid = lax.broadcasted_iota(_i32, (NPAD, 1), 0) < N_NODES
    sp = jnp.maximum(x, 0.0) + jnp.log(1.0 + jnp.exp(-jnp.abs(x)))
    mf_ref[0, 0] = jnp.sum(jnp.where(valid, sp, 0.0)) * (1.0 / N_NODES)
    reg = jnp.sum(um * um) + jnp.sum(p * p) + jnp.sum(n * n)
    el_ref[0, 0] = reg * (0.5 * DECAY / N_NODES)


_loss_tc = pl.pallas_call(
    _loss_body,
    out_shape=(jax.ShapeDtypeStruct((1, 1), _f32),
               jax.ShapeDtypeStruct((1, 1), _f32)),
    out_specs=(pl.BlockSpec(memory_space=pltpu.SMEM),
               pl.BlockSpec(memory_space=pltpu.SMEM)),
)


def kernel(user_table, item_table, edge_index_iu, edge_index_ui,
           pos_items_ur, neg_items_ur):
    pad = NPAD - N_NODES
    u0 = jnp.pad(user_table, ((0, pad), (0, 0)))
    i0 = jnp.pad(item_table, ((0, pad), (0, 0)))

    lists_iu, cnt_iu = _partition(edge_index_iu[0], edge_index_iu[1])
    lists_ui, cnt_ui = _partition(edge_index_ui[0], edge_index_ui[1])
    lists_iu = lists_iu.reshape(NW, LROWS * 16)
    lists_ui = lists_ui.reshape(NW, LROWS * 16)

    u1 = _conv(i0, u0, lists_iu, cnt_iu)
    i1 = _conv(u0, i0, lists_ui, cnt_ui)
    u2 = _conv(i1, u1, lists_iu, cnt_iu)
    i2 = _conv(u1, i1, lists_ui, cnt_ui)

    imean = _mean3(i0, i1, i2)
    fill = jnp.full((pad,), N_NODES, _i32)
    posp = jnp.concatenate([pos_items_ur, fill])
    negp = jnp.concatenate([neg_items_ur, fill])
    pg, ng = _pair_gather(imean, posp, negp)

    mf, el = _loss_tc(u0, u1, u2, pg, ng)
    return (mf[0, 0], el[0, 0], jnp.asarray(0.0, dtype=_f32))


# speedup vs baseline: 3.7673x; 1.0003x over previous
import functools

import jax
import jax.numpy as jnp
from jax import lax
from jax.experimental import pallas as pl
from jax.experimental.pallas import tpu as pltpu
from jax.experimental.pallas import tpu_sc as plsc

N_NODES = 10000
D = 128
NE = 320000
NC = 2
NS = 16
NW = NC * NS
NPAD = 10240
RPW = NPAD // NW
EB = 32
EB_SH = EB.bit_length() - 1
LROWS = 680
SB = 3200
CH = 64
DECAY = 1e-4
EPS = 1e-16

_mesh = plsc.VectorSubcoreMesh(core_axis_name="c", subcore_axis_name="s")

_f32 = jnp.float32
_i32 = jnp.int32

_GDN = lax.GatherDimensionNumbers(
    offset_dims=(), collapsed_slice_dims=(0,), start_index_map=(0,))


def _lperm(x, idx):
    return lax.gather(x, idx.reshape(16, 1), _GDN, slice_sizes=(1,),
                      mode=lax.GatherScatterMode.PROMISE_IN_BOUNDS)


def _hsum(x):
    lanes = lax.iota(_i32, 16)
    for sh in (8, 4, 2, 1):
        x = x + _lperm(x, lanes ^ sh)
    return x


def _wid():
    return lax.axis_index("s") * NC + lax.axis_index("c")


@functools.partial(
    pl.kernel,
    out_type=(jax.ShapeDtypeStruct((NW, LROWS, 16), _i32),
              jax.ShapeDtypeStruct((NW, 16), _i32)),
    mesh=_mesh,
    scratch_types=[
        pltpu.VMEM((SB,), _i32),
        pltpu.VMEM((SB,), _i32),
        pltpu.VMEM((LROWS, 16), _i32),
        pltpu.VMEM((16,), _i32),
    ],
)
def _partition(src_idx, dst_idx, lists, cnt, sstage, dstage, pbuf, cvec):
    wid = _wid()
    lo = wid * RPW
    hi = lo + RPW
    lanes = lax.iota(_i32, 16)
    one_i = jnp.full((16,), 1, _i32)
    zero_i = jnp.full((16,), 0, _i32)
    last = jnp.full((16,), 15, _i32)
    pad_row = jnp.full((16,), hi, _i32)

    def fill(i, carry):
        pbuf[i, pl.ds(0, 16)] = pad_row
        return carry

    lax.fori_loop(0, LROWS, fill, 0)

    def step(it, state):
        off = it * SB
        pltpu.sync_copy(src_idx.at[pl.ds(off, SB)], sstage)
        pltpu.sync_copy(dst_idx.at[pl.ds(off, SB)], dstage)

        def group(j, st):
            r, f, pend = st
            sl = pl.ds(j * 16, 16)
            dg = dstage[sl]
            sg = sstage[sl]
            m = (dg >= lo) & (dg < hi)
            rank = jnp.where(m, one_i, zero_i)
            for sh in (1, 2, 4, 8):
                shifted = _lperm(rank, jnp.maximum(lanes - sh, 0))
                rank = rank + jnp.where(lanes >= sh, shifted, zero_i)
            target = lanes + 1
            sel = zero_i
            for bit in (8, 4, 2, 1):
                probe = _lperm(rank, sel + (bit - 1))
                sel = jnp.where(probe >= target, sel, sel + bit)
            packed = (sg << 14) | dg
            comp = _lperm(packed, sel)
            npop = rank[15]
            merge_a = jnp.where(lanes < f, pend,
                                _lperm(comp, jnp.maximum(lanes - f, 0)))
            merge_b = _lperm(comp, jnp.minimum(lanes + (16 - f), last))
            pbuf[r, pl.ds(0, 16)] = merge_a
            pbuf[r + 1, pl.ds(0, 16)] = merge_b
            tot = f + npop
            roll = tot >> 4
            rollv = jnp.full((16,), roll, _i32)
            pend = merge_a * (1 - rollv) + merge_b * rollv
            return (r + roll, tot & 15, pend)

        return lax.fori_loop(0, SB // 16, group, state)

    r, f, _ = lax.fori_loop(0, NE // SB, step,
                            (jnp.asarray(0, _i32), jnp.asarray(0, _i32),
                             pad_row))
    pbuf[r, pl.ds(0, 16)] = jnp.where(lanes < f,
                                      pbuf[r, pl.ds(0, 16)], pad_row)
    pbuf[r + 1, pl.ds(0, 16)] = pad_row
    pltpu.sync_copy(pbuf, lists.at[wid])
    cvec[...] = jnp.full((16,), r * 16 + f, _i32)
    pltpu.sync_copy(cvec, cnt.at[wid])


@functools.partial(
    pl.kernel,
    out_type=jax.ShapeDtypeStruct((NPAD, D), _f32),
    mesh=_mesh,
    scratch_types=[
        pltpu.VMEM((EB,), _i32),
        pltpu.VMEM((EB,), _i32),
        pltpu.VMEM((EB,), _i32),
        pltpu.VMEM((EB, D), _f32),
        pltpu.VMEM((RPW + 8, D), _f32),
        pltpu.VMEM((RPW + 8, D), _f32),
        pltpu.VMEM((RPW + 8, 16), _f32),
        pltpu.VMEM((16,), _i32),
        pltpu.SemaphoreType.DMA,
    ],
)
def _conv(src_tab, dst_tab, lists, cnt, out,
          pk, lbuf, sidx, srows, dloc, num, den, cvec, sem):
    wid = _wid()
    lo = wid * RPW
    zero16 = jnp.full((16,), 0.0, _f32)

    pltpu.sync_copy(dst_tab.at[pl.ds(lo, RPW)], dloc.at[pl.ds(0, RPW)])

    def zrow(r, carry):
        for q in range(8):
            num[r, pl.ds(q * 16, 16)] = zero16
        den[r, pl.ds(0, 16)] = zero16
        return carry

    lax.fori_loop(0, RPW + 8, zrow, 0)
    for r in range(8):
        for q in range(8):
            dloc[RPW + r, pl.ds(q * 16, 16)] = zero16

    pltpu.sync_copy(cnt.at[wid], cvec)
    cvec[...] = (cvec[...] + (EB - 1)) >> EB_SH
    nblk = cvec[...][0]

    def edge(k, lr, sv):
        acc = sv[0] * dloc[lr, pl.ds(0, 16)]
        for q in range(1, 8):
            acc += sv[q] * dloc[lr, pl.ds(q * 16, 16)]
        e = _hsum(acc)
        e = jnp.where(e >= 0.0, e, 0.2 * e)
        w = jnp.exp(e)
        for q in range(8):
            sl = pl.ds(q * 16, 16)
            num[lr, sl] = num[lr, sl] + w * sv[q]
        den[lr, pl.ds(0, 16)] = den[lr, pl.ds(0, 16)] + w

    def block(b, carry):
        base = b * EB
        pltpu.sync_copy(lists.at[wid, pl.ds(base, EB)], pk)
        for g in range(EB // 16):
            p = pk[pl.ds(g * 16, 16)]
            sidx[pl.ds(g * 16, 16)] = p >> 14
            lbuf[pl.ds(g * 16, 16)] = (p & 16383) - lo
        pltpu.async_copy(src_tab.at[sidx], srows, sem).wait()
        for g in range(EB // 16):
            lg = lbuf[pl.ds(g * 16, 16)]
            for j in range(16):
                k = g * 16 + j
                edge(k, lg[j],
                     [srows[k, pl.ds(q * 16, 16)] for q in range(8)])
        return carry

    lax.fori_loop(0, nblk, block, 0)

    def nrow(r, carry):
        inv = 1.0 / (den[r, pl.ds(0, 16)] + EPS)
        for q in range(8):
            sl = pl.ds(q * 16, 16)
            num[r, sl] = num[r, sl] * inv
        return carry

    lax.fori_loop(0, RPW, nrow, 0)
    pltpu.sync_copy(num.at[pl.ds(0, RPW)], out.at[pl.ds(lo, RPW)])


@functools.partial(
    pl.kernel,
    out_type=jax.ShapeDtypeStruct((NPAD, D), _f32),
    mesh=_mesh,
    scratch_types=[
        pltpu.VMEM((CH, D), _f32),
        pltpu.VMEM((CH, D), _f32),
        pltpu.VMEM((CH, D), _f32),
    ],
)
def _mean3(a, b, c_tab, out, ta, tb, tc):
    wid = _wid()
    third = jnp.full((16,), 1.0 / 3.0, _f32)

    def chunk(ch, carry):
        base = wid * RPW + ch * CH
        pltpu.sync_copy(a.at[pl.ds(base, CH)], ta)
        pltpu.sync_copy(b.at[pl.ds(base, CH)], tb)
        pltpu.sync_copy(c_tab.at[pl.ds(base, CH)], tc)

        def row(r, c2):
            for q in range(8):
                sl = pl.ds(q * 16, 16)
                ta[r, sl] = (ta[r, sl] + tb[r, sl] + tc[r, sl]) * third
            return c2

        lax.fori_loop(0, CH, row, 0)
        pltpu.sync_copy(ta, out.at[pl.ds(base, CH)])
        return carry

    lax.fori_loop(0, RPW // CH, chunk, 0)


@functools.partial(
    pl.kernel,
    out_type=(jax.ShapeDtypeStruct((NPAD, D), _f32),
              jax.ShapeDtypeStruct((NPAD, D), _f32)),
    mesh=_mesh,
    scratch_types=[
        pltpu.VMEM((CH,), _i32),
        pltpu.VMEM((CH,), _i32),
        pltpu.VMEM((CH, D), _f32),
        pltpu.VMEM((CH, D), _f32),
        pltpu.SemaphoreType.DMA,
        pltpu.SemaphoreType.DMA,
    ],
)
def _pair_gather(imean, posp, negp, pg, ng, idxp, idxn, prow, nrow, semp, semn):
    wid = _wid()

    def chunk(ch, carry):
        base = wid * RPW + ch * CH
        pltpu.sync_copy(posp.at[pl.ds(base, CH)], idxp)
        pltpu.sync_copy(negp.at[pl.ds(base, CH)], idxn)
        cp = pltpu.async_copy(imean.at[idxp], prow, semp)
        cn = pltpu.async_copy(imean.at[idxn], nrow, semn)
        cp.wait()
        cn.wait()
        pltpu.sync_copy(prow, pg.at[pl.ds(base, CH)])
        pltpu.sync_copy(nrow, ng.at[pl.ds(base, CH)])
        return carry

    lax.fori_loop(0, RPW // CH, chunk, 0)


def _loss_body(u0, u1, u2, pg, ng, mf_ref, el_ref):
    um = (u0[...] + u1[...] + u2[...]) * (1.0 / 3.0)
    p = pg[...]
    n = ng[...]
    ps = jnp.sum(um * p, axis=1, keepdims=True)
    ns = jnp.sum(um * n, axis=1, keepdims=True)
    x = ns - ps
    valid = lax.broadcasted_iota(_i32, (NPAD, 1), 0) < N_NODES
    sp = jnp.maximum(x, 0.0) + jnp.log(1.0 + jnp.exp(-jnp.abs(x)))
    mf_ref[0, 0] = jnp.sum(jnp.where(valid, sp, 0.0)) * (1.0 / N_NODES)
    reg = jnp.sum(um * um) + jnp.sum(p * p) + jnp.sum(n * n)
    el_ref[0, 0] = reg * (0.5 * DECAY / N_NODES)


_loss_tc = pl.pallas_call(
    _loss_body,
    out_shape=(jax.ShapeDtypeStruct((1, 1), _f32),
               jax.ShapeDtypeStruct((1, 1), _f32)),
    out_specs=(pl.BlockSpec(memory_space=pltpu.SMEM),
               pl.BlockSpec(memory_space=pltpu.SMEM)),
)


def kernel(user_table, item_table, edge_index_iu, edge_index_ui,
           pos_items_ur, neg_items_ur):
    pad = NPAD - N_NODES
    u0 = jnp.pad(user_table, ((0, pad), (0, 0)))
    i0 = jnp.pad(item_table, ((0, pad), (0, 0)))

    lists_iu, cnt_iu = _partition(edge_index_iu[0], edge_index_iu[1])
    lists_ui, cnt_ui = _partition(edge_index_ui[0], edge_index_ui[1])
    lists_iu = lists_iu.reshape(NW, LROWS * 16)
    lists_ui = lists_ui.reshape(NW, LROWS * 16)

    u1 = _conv(i0, u0, lists_iu, cnt_iu)
    i1 = _conv(u0, i0, lists_ui, cnt_ui)
    u2 = _conv(i1, u1, lists_iu, cnt_iu)
    i2 = _conv(u1, i1, lists_ui, cnt_ui)

    imean = _mean3(i0, i1, i2)
    fill = jnp.full((pad,), N_NODES, _i32)
    posp = jnp.concatenate([pos_items_ur, fill])
    negp = jnp.concatenate([neg_items_ur, fill])
    pg, ng = _pair_gather(imean, posp, negp)

    mf, el = _loss_tc(u0, u1, u2, pg, ng)
    return (mf[0, 0], el[0, 0], jnp.asarray(0.0, dtype=_f32))
